# baseline (speedup 1.0000x reference)
import functools

import jax
import jax.numpy as jnp
from jax import lax
from jax.experimental import pallas as pl
from jax.experimental.pallas import tpu as pltpu
from jax.experimental.pallas import tpu_sc as plsc


def _pos_embed_sc(b, h, w, d, dtype):
    info = plsc.get_sparse_core_info()
    nc, ns, lanes = info.num_cores, info.num_subcores, info.num_lanes
    nw = nc * ns
    n_ch = 2 * d
    cpw = n_ch // nw
    assert n_ch % nw == 0 and d % cpw == 0
    assert w % lanes == 0 and h % lanes == 0
    mesh = plsc.VectorSubcoreMesh(core_axis_name="c", subcore_axis_name="s")

    @functools.partial(
        pl.kernel,
        mesh=mesh,
        out_type=jax.ShapeDtypeStruct((b, n_ch, h, w), dtype),
        scratch_types=[
            pltpu.VMEM((max(h, w) * d,), dtype),
            pltpu.VMEM((cpw, h, w), dtype),
            pltpu.SemaphoreType.DMA,
        ],
        compiler_params=pltpu.CompilerParams(
            needs_layout_passes=False, skip_device_barrier=True
        ),
    )
    def k(row_hbm, col_hbm, out_hbm, table_v, plane_v, sem):
        wid = lax.axis_index("s") * nc + lax.axis_index("c")
        ch0 = wid * cpw

        @pl.when(wid < nw // 2)
        def _col_planes():
            pltpu.async_copy(
                col_hbm.at[pl.ds(0, w * d)], table_v.at[pl.ds(0, w * d)], sem
            ).wait()
            for cc in range(cpw):
                c_idx = jnp.full((lanes,), ch0 + cc, dtype=jnp.int32)
                for jh in range(w // lanes):
                    j_idx = jnp.arange(lanes, dtype=jnp.int32) + jh * lanes
                    v = plsc.load_gather(table_v, [j_idx * d + c_idx])
                    for i in range(h):
                        plane_v[cc, i, pl.ds(jh * lanes, lanes)] = v

        @pl.when(wid >= nw // 2)
        def _row_planes():
            pltpu.async_copy(
                row_hbm.at[pl.ds(0, h * d)], table_v.at[pl.ds(0, h * d)], sem
            ).wait()
            for cc in range(cpw):
                c_idx = jnp.full((lanes,), ch0 - d + cc, dtype=jnp.int32)
                for i in range(h):
                    v = plsc.load_gather(table_v, [c_idx + i * d])
                    for jh in range(w // lanes):
                        plane_v[cc, i, pl.ds(jh * lanes, lanes)] = v

        copies = [
            pltpu.async_copy(plane_v, out_hbm.at[bb, pl.ds(ch0, cpw)], sem)
            for bb in range(b)
        ]
        for cp in copies:
            cp.wait()

    return k


def kernel(x, row_embed, col_embed):
    b = x.shape[0]
    h, w = x.shape[-2], x.shape[-1]
    d = row_embed.shape[-1]
    k = _pos_embed_sc(b, h, w, d, row_embed.dtype)
    return k(row_embed.reshape(-1), col_embed.reshape(-1))


# speedup vs baseline: 1.0099x; 1.0099x over previous
import functools

import jax
import jax.numpy as jnp
from jax import lax
from jax.experimental import pallas as pl
from jax.experimental.pallas import tpu as pltpu
from jax.experimental.pallas import tpu_sc as plsc


def _pos_embed_sc(b, h, w, d, dtype):
    info = plsc.get_sparse_core_info()
    nc, ns, lanes = info.num_cores, info.num_subcores, info.num_lanes
    nw = nc * ns
    n_ch = 2 * d
    cpw = n_ch // nw
    assert n_ch % nw == 0 and d % cpw == 0
    assert w % lanes == 0 and h % lanes == 0
    mesh = plsc.VectorSubcoreMesh(core_axis_name="c", subcore_axis_name="s")

    @functools.partial(
        pl.kernel,
        mesh=mesh,
        out_type=jax.ShapeDtypeStruct((b, n_ch, h, w), dtype),
        scratch_types=[
            pltpu.VMEM((max(h, w) * d,), dtype),
            pltpu.VMEM((cpw, h, w), dtype),
            pltpu.SemaphoreType.DMA,
        ],
        compiler_params=pltpu.CompilerParams(
            needs_layout_passes=False, skip_device_barrier=True
        ),
    )
    def k(row_hbm, col_hbm, out_hbm, table_v, plane_v, sem):
        wid = lax.axis_index("s") * nc + lax.axis_index("c")
        ch0 = wid * cpw

        @pl.when(wid < nw // 2)
        def _col_planes():
            with jax.named_scope("in_dma"):
                pltpu.async_copy(
                    col_hbm.at[pl.ds(0, w * d)],
                    table_v.at[pl.ds(0, w * d)],
                    sem,
                ).wait()
            with jax.named_scope("build_col"):
                for cc in range(cpw):
                    c_idx = jnp.full((lanes,), ch0 + cc, dtype=jnp.int32)
                    for jh in range(w // lanes):
                        j_idx = jnp.arange(lanes, dtype=jnp.int32) + jh * lanes
                        v = plsc.load_gather(table_v, [j_idx * d + c_idx])
                        for i in range(h):
                            plane_v[cc, i, pl.ds(jh * lanes, lanes)] = v

        @pl.when(wid >= nw // 2)
        def _row_planes():
            with jax.named_scope("in_dma"):
                pltpu.async_copy(
                    row_hbm.at[pl.ds(0, h * d)],
                    table_v.at[pl.ds(0, h * d)],
                    sem,
                ).wait()
            with jax.named_scope("build_row"):
                for cc in range(cpw):
                    c_idx = jnp.full((lanes,), ch0 - d + cc, dtype=jnp.int32)
                    for i in range(h):
                        v = plsc.load_gather(table_v, [c_idx + i * d])
                        for jh in range(w // lanes):
                            plane_v[cc, i, pl.ds(jh * lanes, lanes)] = v

        with jax.named_scope("out_dma"):
            copies = [
                pltpu.async_copy(plane_v, out_hbm.at[bb, pl.ds(ch0, cpw)], sem)
                for bb in range(b)
            ]
            for cp in copies:
                cp.wait()

    return k


def kernel(x, row_embed, col_embed):
    b = x.shape[0]
    h, w = x.shape[-2], x.shape[-1]
    d = row_embed.shape[-1]
    k = _pos_embed_sc(b, h, w, d, row_embed.dtype)
    return k(row_embed.reshape(-1), col_embed.reshape(-1))


# speedup vs baseline: 1.0103x; 1.0004x over previous
import functools

import jax
import jax.numpy as jnp
from jax import lax
from jax.experimental import pallas as pl
from jax.experimental.pallas import tpu as pltpu
from jax.experimental.pallas import tpu_sc as plsc


def _pos_embed_sc(b, h, w, d, dtype):
    info = plsc.get_sparse_core_info()
    nc, ns, lanes = info.num_cores, info.num_subcores, info.num_lanes
    nw = nc * ns
    n_ch = 2 * d
    cpw = n_ch // nw
    assert n_ch % nw == 0 and d % cpw == 0
    assert w % lanes == 0 and h % lanes == 0
    mesh = plsc.VectorSubcoreMesh(core_axis_name="c", subcore_axis_name="s")

    @functools.partial(
        pl.kernel,
        mesh=mesh,
        out_type=jax.ShapeDtypeStruct((b, n_ch, h, w), dtype),
        scratch_types=[
            pltpu.VMEM((max(h, w) * d,), dtype),
            pltpu.VMEM((cpw, h, w), dtype),
            pltpu.SemaphoreType.DMA,
        ],
        compiler_params=pltpu.CompilerParams(
            needs_layout_passes=False,
            skip_device_barrier=True,
            use_tc_tiling_on_sc=True,
        ),
    )
    def k(row_hbm, col_hbm, out_hbm, table_v, plane_v, sem):
        wid = lax.axis_index("s") * nc + lax.axis_index("c")
        ch0 = wid * cpw

        @pl.when(wid < nw // 2)
        def _col_planes():
            with jax.named_scope("in_dma"):
                pltpu.async_copy(
                    col_hbm.at[pl.ds(0, w * d)],
                    table_v.at[pl.ds(0, w * d)],
                    sem,
                ).wait()
            with jax.named_scope("build_col"):
                for cc in range(cpw):
                    c_idx = jnp.full((lanes,), ch0 + cc, dtype=jnp.int32)
                    for jh in range(w // lanes):
                        j_idx = jnp.arange(lanes, dtype=jnp.int32) + jh * lanes
                        v = plsc.load_gather(table_v, [j_idx * d + c_idx])
                        for i in range(h):
                            plane_v[cc, i, pl.ds(jh * lanes, lanes)] = v

        @pl.when(wid >= nw // 2)
        def _row_planes():
            with jax.named_scope("in_dma"):
                pltpu.async_copy(
                    row_hbm.at[pl.ds(0, h * d)],
                    table_v.at[pl.ds(0, h * d)],
                    sem,
                ).wait()
            with jax.named_scope("build_row"):
                for cc in range(cpw):
                    c_idx = jnp.full((lanes,), ch0 - d + cc, dtype=jnp.int32)
                    for i in range(h):
                        v = plsc.load_gather(table_v, [c_idx + i * d])
                        for jh in range(w // lanes):
                            plane_v[cc, i, pl.ds(jh * lanes, lanes)] = v

        with jax.named_scope("out_dma"):
            copies = [
                pltpu.async_copy(plane_v, out_hbm.at[bb, pl.ds(ch0, cpw)], sem)
                for bb in range(b)
            ]
            for cp in copies:
                cp.wait()

    return k


def kernel(x, row_embed, col_embed):
    b = x.shape[0]
    h, w = x.shape[-2], x.shape[-1]
    d = row_embed.shape[-1]
    k = _pos_embed_sc(b, h, w, d, row_embed.dtype)
    return k(row_embed.reshape(-1), col_embed.reshape(-1))


# speedup vs baseline: 1.3539x; 1.3400x over previous
import functools

import jax
import jax.numpy as jnp
from jax import lax
from jax.experimental import pallas as pl
from jax.experimental.pallas import tpu as pltpu
from jax.experimental.pallas import tpu_sc as plsc


def _pos_embed_sc(b, h, w, d, dtype):
    info = plsc.get_sparse_core_info()
    nc, ns, lanes = info.num_cores, info.num_subcores, info.num_lanes
    nw = nc * ns
    n_ch = 2 * d
    cpw = n_ch // nw
    assert n_ch % nw == 0 and d % cpw == 0
    assert w % lanes == 0 and h % lanes == 0
    mesh = plsc.VectorSubcoreMesh(core_axis_name="c", subcore_axis_name="s")

    @functools.partial(
        pl.kernel,
        mesh=mesh,
        out_type=jax.ShapeDtypeStruct((b, n_ch, h * w), dtype),
        scratch_types=[
            pltpu.VMEM((max(h, w) * d,), dtype),
            pltpu.VMEM((cpw, h * w), dtype),
            pltpu.SemaphoreType.DMA,
        ],
        compiler_params=pltpu.CompilerParams(
            needs_layout_passes=False,
            skip_device_barrier=True,
            use_tc_tiling_on_sc=True,
        ),
    )
    def k(row_hbm, col_hbm, out_hbm, table_v, plane_v, sem):
        wid = lax.axis_index("s") * nc + lax.axis_index("c")
        ch0 = wid * cpw

        @pl.when(wid < nw // 2)
        def _col_planes():
            with jax.named_scope("in_dma"):
                pltpu.async_copy(
                    col_hbm.at[pl.ds(0, w * d)],
                    table_v.at[pl.ds(0, w * d)],
                    sem,
                ).wait()
            with jax.named_scope("build_col"):
                for cc in range(cpw):
                    c_idx = jnp.full((lanes,), ch0 + cc, dtype=jnp.int32)
                    for jh in range(w // lanes):
                        j_idx = jnp.arange(lanes, dtype=jnp.int32) + jh * lanes
                        v = plsc.load_gather(table_v, [j_idx * d + c_idx])
                        for i in range(h):
                            plane_v[cc, pl.ds(i * w + jh * lanes, lanes)] = v

        @pl.when(wid >= nw // 2)
        def _row_planes():
            with jax.named_scope("in_dma"):
                pltpu.async_copy(
                    row_hbm.at[pl.ds(0, h * d)],
                    table_v.at[pl.ds(0, h * d)],
                    sem,
                ).wait()
            with jax.named_scope("build_row"):
                for cc in range(cpw):
                    c_idx = jnp.full((lanes,), ch0 - d + cc, dtype=jnp.int32)
                    for i in range(h):
                        v = plsc.load_gather(table_v, [c_idx + i * d])
                        for jh in range(w // lanes):
                            plane_v[cc, pl.ds(i * w + jh * lanes, lanes)] = v

        with jax.named_scope("out_dma"):
            copies = [
                pltpu.async_copy(plane_v, out_hbm.at[bb, pl.ds(ch0, cpw)], sem)
                for bb in range(b)
            ]
            for cp in copies:
                cp.wait()

    return k


def kernel(x, row_embed, col_embed):
    b = x.shape[0]
    h, w = x.shape[-2], x.shape[-1]
    d = row_embed.shape[-1]
    k = _pos_embed_sc(b, h, w, d, row_embed.dtype)
    out = k(row_embed.reshape(-1), col_embed.reshape(-1))
    return out.reshape(b, 2 * d, h, w)
